# baseline (speedup 1.0000x reference)
import functools

import jax
import jax.numpy as jnp
from jax import lax
from jax.experimental import pallas as pl
from jax.experimental.pallas import tpu as pltpu
from jax.experimental.pallas import tpu_sc as plsc

D_MODEL = 1024
SCALE = 32.0
NC = 2
NS = 16
NW = NC * NS
LANES = 16
C = 16
RING = 6
LEAD = 4


@functools.partial(jax.jit, static_argnums=(2,))
def _emb(idx, table, B):
    chunks = B // (NW * C)
    mesh = plsc.VectorSubcoreMesh(core_axis_name="c", subcore_axis_name="s")

    @functools.partial(
        pl.kernel,
        out_type=jax.ShapeDtypeStruct((B, D_MODEL), jnp.float32),
        mesh=mesh,
        scratch_types=(
            [pltpu.VMEM((chunks, C), jnp.int32)]
            + [pltpu.VMEM((C, D_MODEL), jnp.float32)] * RING
            + [pltpu.SemaphoreType.DMA] * (2 * RING)
        ),
    )
    def emb_kernel(idx_hbm, table_hbm, out_hbm, idx_v, *bufs_and_sems):
        bufs = bufs_and_sems[:RING]
        sis = bufs_and_sems[RING:2 * RING]
        sos = bufs_and_sems[2 * RING:]
        wid = lax.axis_index("s") * NC + lax.axis_index("c")
        base = wid * (chunks * C)
        pltpu.sync_copy(idx_hbm.at[wid], idx_v)
        for b in range(LEAD):
            pltpu.async_copy(table_hbm.at[idx_v.at[b]], bufs[b], sis[b])

        def step(j, b, regather):
            buf, si, so = bufs[b], sis[b], sos[b]
            pltpu.make_async_copy(table_hbm.at[idx_v.at[j]], buf, si).wait()

            @plsc.parallel_loop(0, C, 1)
            def row_body(r):
                for k in range(D_MODEL // LANES):
                    sl = pl.ds(k * LANES, LANES)
                    buf[r, sl] = buf[r, sl] * SCALE

            pltpu.async_copy(buf, out_hbm.at[pl.ds(base + j * C, C)], so)

            if regather:
                bg = (b + LEAD) % RING

                @pl.when(j >= RING - LEAD)
                def _():
                    pltpu.make_async_copy(
                        bufs[bg], out_hbm.at[pl.ds(base, C)], sos[bg]).wait()
                pltpu.async_copy(table_hbm.at[idx_v.at[j + LEAD]], bufs[bg],
                                 sis[bg])

        def outer(jj, carry):
            for u in range(RING):
                step(RING * jj + u, u, regather=True)
            return carry

        n_main = ((chunks - LEAD) // RING) * RING
        lax.fori_loop(0, (chunks - LEAD) // RING, outer, 0)
        for j in range(n_main, chunks):
            step(j, j % RING, regather=j < chunks - LEAD)
        for j in range(chunks - RING, chunks):
            b = j % RING
            pltpu.make_async_copy(
                bufs[b], out_hbm.at[pl.ds(base + j * C, C)], sos[b]).wait()

    return emb_kernel(idx, table)


def kernel(x, table):
    b, s = x.shape
    B = b * s
    idx = x.reshape(NW, B // (NW * C), C).astype(jnp.int32)
    out = _emb(idx, table, B)
    return out.reshape(b, s, D_MODEL)


# speedup vs baseline: 1.0300x; 1.0300x over previous
import functools

import jax
import jax.numpy as jnp
from jax import lax
from jax.experimental import pallas as pl
from jax.experimental.pallas import tpu as pltpu
from jax.experimental.pallas import tpu_sc as plsc

D_MODEL = 1024
SCALE = 32.0
NC = 2
NS = 16
NW = NC * NS
LANES = 16
C = 16
NIN = 4
NOUT = 2
UNROLL = 4


@functools.partial(jax.jit, static_argnums=(2,))
def _emb(idx, table, B):
    chunks = B // (NW * C)
    mesh = plsc.VectorSubcoreMesh(core_axis_name="c", subcore_axis_name="s")

    @functools.partial(
        pl.kernel,
        out_type=jax.ShapeDtypeStruct((B, D_MODEL), jnp.float32),
        mesh=mesh,
        scratch_types=(
            [pltpu.VMEM((chunks, C), jnp.int32)]
            + [pltpu.VMEM((C, D_MODEL), jnp.float32)] * (NIN + NOUT)
            + [pltpu.SemaphoreType.DMA] * (NIN + NOUT)
        ),
    )
    def emb_kernel(idx_hbm, table_hbm, out_hbm, idx_v, *bufs_and_sems):
        ins = bufs_and_sems[:NIN]
        outs = bufs_and_sems[NIN:NIN + NOUT]
        sis = bufs_and_sems[NIN + NOUT:2 * NIN + NOUT]
        sos = bufs_and_sems[2 * NIN + NOUT:]
        wid = lax.axis_index("s") * NC + lax.axis_index("c")
        base = wid * (chunks * C)
        pltpu.sync_copy(idx_hbm.at[wid], idx_v)
        for b in range(NIN):
            pltpu.async_copy(table_hbm.at[idx_v.at[b]], ins[b], sis[b])

        def step(j, b, ob, regather):
            inb, sib = ins[b], sis[b]
            outb, sob = outs[ob], sos[ob]
            pltpu.make_async_copy(table_hbm.at[idx_v.at[j]], inb, sib).wait()

            @pl.when(j >= NOUT)
            def _():
                pltpu.make_async_copy(
                    outb, out_hbm.at[pl.ds(base, C)], sob).wait()

            @plsc.parallel_loop(0, C, 1)
            def row_body(r):
                for k in range(D_MODEL // LANES):
                    sl = pl.ds(k * LANES, LANES)
                    outb[r, sl] = inb[r, sl] * SCALE

            if regather:
                @pl.when(j < chunks - NIN)
                def _():
                    pltpu.async_copy(table_hbm.at[idx_v.at[j + NIN]], inb, sib)

            pltpu.async_copy(outb, out_hbm.at[pl.ds(base + j * C, C)], sob)

        def outer(jj, carry):
            for u in range(UNROLL):
                step(UNROLL * jj + u, u % NIN, u % NOUT, regather=True)
            return carry

        n_main = (chunks // UNROLL) * UNROLL
        lax.fori_loop(0, chunks // UNROLL, outer, 0)
        for j in range(n_main, chunks):
            step(j, j % NIN, j % NOUT, regather=False)
        for u in range(NOUT):
            j = chunks - NOUT + u
            pltpu.make_async_copy(
                outs[j % NOUT], out_hbm.at[pl.ds(base + j * C, C)],
                sos[j % NOUT]).wait()

    return emb_kernel(idx, table)


def kernel(x, table):
    b, s = x.shape
    B = b * s
    idx = x.reshape(NW, B // (NW * C), C).astype(jnp.int32)
    out = _emb(idx, table, B)
    return out.reshape(b, s, D_MODEL)
